# Initial kernel scaffold; baseline (speedup 1.0000x reference)
import jax
import jax.numpy as jnp
from jax.experimental import pallas as pl


def kernel(y, e, mu_causal, cov_causal, mu_spurious, cov_spurious):
    raise NotImplementedError("write your pallas kernel here")


# speedup vs baseline: 1.0152x; 1.0152x over previous
import functools

import jax
import jax.numpy as jnp
from jax import lax
from jax.experimental import pallas as pl
from jax.experimental.pallas import tpu as pltpu
from jax.experimental.pallas import tpu_sc as plsc

N_ENVS = 100
N_CLASSES = 1000
Z = 64
BATCH = 4096

_info = plsc.get_sparse_core_info()
_NC, _NS, _L = _info.num_cores, _info.num_subcores, _info.num_lanes
_NW = _NC * _NS
_BPW = BATCH // _NW


def _sc_gather_body(y_hbm, e_hbm, mu_c_hbm, cov_c_hbm, mu_s_hbm, cov_s_hbm,
                    muc_out, mus_out, covc_out, covs_out,
                    y_v, e_v, flat_v, muc_v, covc_v, mus_v, covs_v, sem):
    wid = lax.axis_index("s") * _NC + lax.axis_index("c")
    base = wid * _BPW
    pltpu.sync_copy(y_hbm.at[pl.ds(base, _BPW)], y_v)
    pltpu.sync_copy(e_hbm.at[pl.ds(base, _BPW)], e_v)
    for j in range(_BPW // _L):
        sl = pl.ds(j * _L, _L)
        flat_v[sl] = y_v[sl] * N_ENVS + e_v[sl]
    c1 = pltpu.make_async_copy(mu_c_hbm.at[e_v], muc_v, sem)
    c2 = pltpu.make_async_copy(cov_c_hbm.at[e_v], covc_v, sem)
    c3 = pltpu.make_async_copy(mu_s_hbm.at[flat_v], mus_v, sem)
    c4 = pltpu.make_async_copy(cov_s_hbm.at[flat_v], covs_v, sem)
    c1.start(); c2.start(); c3.start(); c4.start()
    c1.wait(); c2.wait(); c3.wait(); c4.wait()
    rows = pl.ds(base, _BPW)
    pltpu.sync_copy(muc_v, muc_out.at[rows])
    pltpu.sync_copy(mus_v, mus_out.at[rows])
    pltpu.sync_copy(covc_v, covc_out.at[rows])
    pltpu.sync_copy(covs_v, covs_out.at[rows])


_sc_gather = functools.partial(
    pl.kernel,
    mesh=plsc.VectorSubcoreMesh(core_axis_name="c", subcore_axis_name="s"),
    out_type=[jax.ShapeDtypeStruct((BATCH, Z), jnp.float32)] * 4,
    scratch_types=[
        pltpu.VMEM((_BPW,), jnp.int32),
        pltpu.VMEM((_BPW,), jnp.int32),
        pltpu.VMEM((_BPW,), jnp.int32),
        pltpu.VMEM((_BPW, Z), jnp.float32),
        pltpu.VMEM((_BPW, Z), jnp.float32),
        pltpu.VMEM((_BPW, Z), jnp.float32),
        pltpu.VMEM((_BPW, Z), jnp.float32),
        pltpu.SemaphoreType.DMA,
    ],
    compiler_params=pltpu.CompilerParams(use_tc_tiling_on_sc=False),
)(_sc_gather_body)


_BB = 128


def _tc_body(muc_ref, mus_ref, covc_ref, covs_ref, mu_ref, out_ref):
    mu_ref[...] = jnp.concatenate([muc_ref[...], mus_ref[...]], axis=-1)
    cov = jax.nn.softplus(
        jnp.concatenate([covc_ref[...], covs_ref[...]], axis=-1))
    i = lax.broadcasted_iota(jnp.int32, (_BB, 2 * Z, 2 * Z), 1)
    j = lax.broadcasted_iota(jnp.int32, (_BB, 2 * Z, 2 * Z), 2)
    out_ref[...] = jnp.where(i == j, cov[:, :, None], jnp.float32(0.0))


def _tc_diag(muc, mus, covc, covs):
    half = pl.BlockSpec((_BB, Z), lambda b: (b, 0))
    return pl.pallas_call(
        _tc_body,
        grid=(BATCH // _BB,),
        in_specs=[half, half, half, half],
        out_specs=[
            pl.BlockSpec((_BB, 2 * Z), lambda b: (b, 0)),
            pl.BlockSpec((_BB, 2 * Z, 2 * Z), lambda b: (b, 0, 0)),
        ],
        out_shape=[
            jax.ShapeDtypeStruct((BATCH, 2 * Z), jnp.float32),
            jax.ShapeDtypeStruct((BATCH, 2 * Z, 2 * Z), jnp.float32),
        ],
    )(muc, mus, covc, covs)


def kernel(y, e, mu_causal, cov_causal, mu_spurious, cov_spurious):
    y_flat = y[:, 0].astype(jnp.int32)
    e_flat = e[:, 0].astype(jnp.int32)
    mu_s2d = mu_spurious.reshape(N_CLASSES * N_ENVS, Z)
    cov_s2d = cov_spurious.reshape(N_CLASSES * N_ENVS, Z)
    muc, mus, covc, covs = _sc_gather(y_flat, e_flat, mu_causal, cov_causal,
                                      mu_s2d, cov_s2d)
    mu, cov_mat = _tc_diag(muc, mus, covc, covs)
    return mu, cov_mat


# speedup vs baseline: 1.0250x; 1.0096x over previous
import functools

import jax
import jax.numpy as jnp
from jax import lax
from jax.experimental import pallas as pl
from jax.experimental.pallas import tpu as pltpu
from jax.experimental.pallas import tpu_sc as plsc

N_ENVS = 100
N_CLASSES = 1000
Z = 64
BATCH = 4096

_info = plsc.get_sparse_core_info()
_NC, _NS, _L = _info.num_cores, _info.num_subcores, _info.num_lanes
_NW = _NC * _NS
_BPW = BATCH // _NW


def _sc_gather_body(y_hbm, e_hbm, mu_c_hbm, cov_c_hbm, mu_s_hbm, cov_s_hbm,
                    muc_out, mus_out, covc_out, covs_out,
                    y_v, e_v, flat_v, muc_v, covc_v, mus_v, covs_v, sem):
    wid = lax.axis_index("s") * _NC + lax.axis_index("c")
    base = wid * _BPW
    pltpu.sync_copy(y_hbm.at[pl.ds(base, _BPW)], y_v)
    pltpu.sync_copy(e_hbm.at[pl.ds(base, _BPW)], e_v)
    for j in range(_BPW // _L):
        sl = pl.ds(j * _L, _L)
        flat_v[sl] = y_v[sl] * N_ENVS + e_v[sl]
    c1 = pltpu.make_async_copy(mu_c_hbm.at[e_v], muc_v, sem)
    c2 = pltpu.make_async_copy(cov_c_hbm.at[e_v], covc_v, sem)
    c3 = pltpu.make_async_copy(mu_s_hbm.at[flat_v], mus_v, sem)
    c4 = pltpu.make_async_copy(cov_s_hbm.at[flat_v], covs_v, sem)
    c1.start(); c2.start(); c3.start(); c4.start()
    c1.wait(); c2.wait(); c3.wait(); c4.wait()
    rows = pl.ds(base, _BPW)
    pltpu.sync_copy(muc_v, muc_out.at[rows])
    pltpu.sync_copy(mus_v, mus_out.at[rows])
    pltpu.sync_copy(covc_v, covc_out.at[rows])
    pltpu.sync_copy(covs_v, covs_out.at[rows])


_sc_gather = functools.partial(
    pl.kernel,
    mesh=plsc.VectorSubcoreMesh(core_axis_name="c", subcore_axis_name="s"),
    out_type=[jax.ShapeDtypeStruct((BATCH, Z), jnp.float32)] * 4,
    scratch_types=[
        pltpu.VMEM((_BPW,), jnp.int32),
        pltpu.VMEM((_BPW,), jnp.int32),
        pltpu.VMEM((_BPW,), jnp.int32),
        pltpu.VMEM((_BPW, Z), jnp.float32),
        pltpu.VMEM((_BPW, Z), jnp.float32),
        pltpu.VMEM((_BPW, Z), jnp.float32),
        pltpu.VMEM((_BPW, Z), jnp.float32),
        pltpu.SemaphoreType.DMA,
    ],
    compiler_params=pltpu.CompilerParams(use_tc_tiling_on_sc=False),
)(_sc_gather_body)


_BB = 256


def _tc_body(muc_ref, mus_ref, covc_ref, covs_ref, mu_ref, out_ref):
    mu_ref[...] = jnp.concatenate([muc_ref[...], mus_ref[...]], axis=-1)
    cov = jax.nn.softplus(
        jnp.concatenate([covc_ref[...], covs_ref[...]], axis=-1))
    eye = (lax.broadcasted_iota(jnp.int32, (2 * Z, 2 * Z), 0)
           == lax.broadcasted_iota(jnp.int32, (2 * Z, 2 * Z), 1))
    out_ref[...] = jnp.where(eye[None], cov[:, :, None], jnp.float32(0.0))


def _tc_diag(muc, mus, covc, covs):
    half = pl.BlockSpec((_BB, Z), lambda b: (b, 0))
    return pl.pallas_call(
        _tc_body,
        grid=(BATCH // _BB,),
        in_specs=[half, half, half, half],
        out_specs=[
            pl.BlockSpec((_BB, 2 * Z), lambda b: (b, 0)),
            pl.BlockSpec((_BB, 2 * Z, 2 * Z), lambda b: (b, 0, 0)),
        ],
        out_shape=[
            jax.ShapeDtypeStruct((BATCH, 2 * Z), jnp.float32),
            jax.ShapeDtypeStruct((BATCH, 2 * Z, 2 * Z), jnp.float32),
        ],
    )(muc, mus, covc, covs)


def kernel(y, e, mu_causal, cov_causal, mu_spurious, cov_spurious):
    y_flat = y[:, 0].astype(jnp.int32)
    e_flat = e[:, 0].astype(jnp.int32)
    mu_s2d = mu_spurious.reshape(N_CLASSES * N_ENVS, Z)
    cov_s2d = cov_spurious.reshape(N_CLASSES * N_ENVS, Z)
    muc, mus, covc, covs = _sc_gather(y_flat, e_flat, mu_causal, cov_causal,
                                      mu_s2d, cov_s2d)
    mu, cov_mat = _tc_diag(muc, mus, covc, covs)
    return mu, cov_mat
